# baseline (speedup 1.0000x reference)
import jax
import jax.numpy as jnp
from jax import lax
from jax.experimental import pallas as pl
from jax.experimental.pallas import tpu as pltpu
from jax.experimental.pallas import tpu_sc as plsc

N = 10000
E = 320000
NCAT = 4
VOCAB = 1000
HID = 128
OUT = 16

NW = 32
PADN = 10240

EB = 64
NBLK = 5120
BPW = NBLK // NW
CHUNK = 40
NCHUNK = BPW // CHUNK

EB2 = 512
NBLK2 = 640
BPW2 = NBLK2 // NW
CHUNK2 = 20
NCHUNK2 = BPW2 // CHUNK2

_mesh = plsc.VectorSubcoreMesh(core_axis_name="c", subcore_axis_name="s")
_f32 = jnp.float32
_bf16 = jnp.bfloat16
_i32 = jnp.int32


def _zero2d(ref, nrows, width):
    @pl.loop(0, nrows)
    def _(r):
        for c in range(width // 16):
            ref[r, pl.ds(c * 16, 16)] = jnp.zeros((16,), _f32)


def _dump_core_slice(sh, hbm_core, sid):
    d0 = sid * 624
    pltpu.sync_copy(sh.at[pl.ds(d0, 624)], hbm_core.at[pl.ds(d0, 624)])

    @pl.when(sid == 15)
    def _():
        pltpu.sync_copy(sh.at[pl.ds(9984, 16)], hbm_core.at[pl.ds(9984, 16)])


def _unpack_rows(src_bf, dst_f32, nrows):
    @pl.loop(0, nrows)
    def _(r):
        for c in range(HID // 32):
            ab = src_bf[r, pl.ds(c * 32, 32)]
            a, b = plsc.unpack(ab, format=plsc.PackFormat.INTERLEAVED)
            dst_f32[r, pl.ds(c * 32, 16)] = a
            dst_f32[r, pl.ds(c * 32 + 16, 16)] = b


def _pack_rows(src_f32, dst_bf, nrows):
    @pl.loop(0, nrows)
    def _(r):
        for c in range(HID // 32):
            a = src_f32[r, pl.ds(c * 32, 16)]
            b = src_f32[r, pl.ds(c * 32 + 16, 16)]
            dst_bf[r, pl.ds(c * 32, 32)] = plsc.pack(
                a, b, format=plsc.PackFormat.INTERLEAVED
            )


def _agg_loop(tbl_hbm, src_hbm, dst_hbm, acc_sh, srcc, dstc, grows, srows,
              gsems, ssems, start, chunk, nchunk, convert):
    @pl.loop(0, nchunk)
    def _(ci):
        c0 = start + ci * chunk
        pltpu.sync_copy(src_hbm.at[pl.ds(c0, chunk)], srcc)
        pltpu.sync_copy(dst_hbm.at[pl.ds(c0, chunk)], dstc)
        pltpu.async_copy(tbl_hbm.at[srcc.at[0]], grows[0], gsems[0])

        @pl.loop(0, chunk // 2)
        def _(pi):
            for half in range(2):
                b = pi * 2 + half
                o = 1 - half

                @pl.when(b >= 1)
                def _():
                    pltpu.make_async_copy(
                        srows[o], acc_sh.at[dstc.at[0]], ssems[o]
                    ).wait()

                @pl.when(b + 1 < chunk)
                def _():
                    pltpu.async_copy(tbl_hbm.at[srcc.at[b + 1]], grows[o], gsems[o])

                pltpu.make_async_copy(
                    tbl_hbm.at[srcc.at[b]], grows[half], gsems[half]
                ).wait()
                if convert is not None:
                    convert(grows[half], srows[half])
                pltpu.async_copy(
                    srows[half], acc_sh.at[dstc.at[b]], ssems[half], add=True
                )

        pltpu.make_async_copy(srows[1], acc_sh.at[dstc.at[0]], ssems[1]).wait()


def _k1_fold(emb_ref, w_ref, f_ref):
    f_ref[0] = jnp.dot(emb_ref[0], w_ref[0], preferred_element_type=_f32)


def _k1_dense(x_ref, w_ref, b_ref, r_ref):
    r_ref[...] = (
        jnp.dot(x_ref[...], w_ref[...], preferred_element_type=_f32) + b_ref[...]
    )


RB = 80
NRB = N // RB


def _k2_body(r_hbm, f_hbm, cat_hbm, dst_hbm, h1p_hbm, h1pbf_hbm, deg_hbm,
             idx0, idx1, acc0, acc1, hbf0, hbf1, dstc, ones_v, z16, deg_sh,
             gsem0, gsem1, wsem0, wsem1, bsem0, bsem1, ssem0, ssem1):
    cid = lax.axis_index("c")
    sid = lax.axis_index("s")
    wid = sid * 2 + cid
    idxs = (idx0, idx1)
    accs = (acc0, acc1)
    hbfs = (hbf0, hbf1)
    gsems = (gsem0, gsem1)
    wsems = (wsem0, wsem1)
    bsems = (bsem0, bsem1)
    ssems = (ssem0, ssem1)

    @pl.loop(0, EB2)
    def _(r):
        ones_v[r, :] = jnp.ones((16,), _f32)

    _zero2d(z16, 128, 16)
    z0 = sid * 640
    for k in range(5):
        pltpu.sync_copy(z16, deg_sh.at[pl.ds(z0 + k * 128, 128)])
    plsc.subcore_barrier()

    nblk = jnp.where(wid < 29, 4, 3)
    start = jnp.where(wid < 29, 4 * wid, 3 * wid + 29)

    for bi in range(5):
        k = bi % 2
        o = (bi - 1) % 2

        if bi < 4:
            @pl.when(bi < nblk)
            def _():
                b = start + bi
                if bi >= 2:
                    pltpu.make_async_copy(
                        accs[k], h1p_hbm.at[pl.ds(0, RB)], wsems[k]
                    ).wait()
                    pltpu.make_async_copy(
                        hbfs[k], h1pbf_hbm.at[pl.ds(0, RB)], bsems[k]
                    ).wait()
                pltpu.sync_copy(
                    cat_hbm.at[pl.ds(b * (NCAT * RB), NCAT * RB)], idxs[k]
                )
                pltpu.sync_copy(r_hbm.at[pl.ds(b * RB, RB)], accs[k])
                for j in range(NCAT):
                    pltpu.async_copy(
                        f_hbm.at[j].at[idxs[k].at[pl.ds(j * RB, RB)]],
                        accs[k], gsems[k], add=True,
                    )

        if bi >= 1:
            @pl.when(bi - 1 < nblk)
            def _():
                bp = start + (bi - 1)
                for j in range(NCAT):
                    pltpu.make_async_copy(
                        f_hbm.at[j].at[idxs[o].at[pl.ds(j * RB, RB)]],
                        accs[o], gsems[o],
                    ).wait()
                _pack_rows(accs[o], hbfs[o], RB)
                pltpu.async_copy(
                    accs[o], h1p_hbm.at[pl.ds(bp * RB, RB)], wsems[o]
                )
                pltpu.async_copy(
                    hbfs[o], h1pbf_hbm.at[pl.ds(bp * RB, RB)], bsems[o]
                )

    estart = wid * BPW2

    @pl.loop(0, NCHUNK2)
    def _(ci):
        pltpu.sync_copy(dst_hbm.at[pl.ds(estart + ci * CHUNK2, CHUNK2)], dstc)

        @pl.loop(0, CHUNK2 // 2)
        def _(pi):
            for half in range(2):
                b = pi * 2 + half

                @pl.when(b >= 2)
                def _():
                    pltpu.make_async_copy(
                        ones_v, deg_sh.at[dstc.at[0]], ssems[half]
                    ).wait()

                pltpu.async_copy(
                    ones_v, deg_sh.at[dstc.at[b]], ssems[half], add=True
                )

        pltpu.make_async_copy(ones_v, deg_sh.at[dstc.at[0]], ssems[0]).wait()
        pltpu.make_async_copy(ones_v, deg_sh.at[dstc.at[0]], ssems[1]).wait()

    pltpu.make_async_copy(acc0, h1p_hbm.at[pl.ds(0, RB)], wsem0).wait()
    pltpu.make_async_copy(acc1, h1p_hbm.at[pl.ds(0, RB)], wsem1).wait()
    pltpu.make_async_copy(hbf0, h1pbf_hbm.at[pl.ds(0, RB)], bsem0).wait()
    pltpu.make_async_copy(hbf1, h1pbf_hbm.at[pl.ds(0, RB)], bsem1).wait()

    plsc.subcore_barrier()
    _dump_core_slice(deg_sh, deg_hbm.at[cid], sid)


def _k3_body(h1pbf_hbm, src_hbm, dst_hbm, agg_hbm,
             srcc, dstc, gr0, gr1, sr0, sr1, acc_sh,
             gsem0, gsem1, ssem0, ssem1):
    cid = lax.axis_index("c")
    sid = lax.axis_index("s")
    wid = sid * 2 + cid

    _zero2d(sr0, EB, HID)
    z0 = sid * 640
    for k in range(640 // EB):
        pltpu.sync_copy(sr0, acc_sh.at[pl.ds(z0 + k * EB, EB)])
    plsc.subcore_barrier()

    def cvt(src_bf, dst_f32):
        _unpack_rows(src_bf, dst_f32, EB)

    _agg_loop(h1pbf_hbm, src_hbm, dst_hbm, acc_sh, srcc, dstc, (gr0, gr1),
              (sr0, sr1), (gsem0, gsem1), (ssem0, ssem1), wid * BPW, CHUNK,
              NCHUNK, cvt)

    plsc.subcore_barrier()
    _dump_core_slice(acc_sh, agg_hbm.at[cid], sid)


def _k5_body(h2_hbm, src_hbm, dst_hbm, agg_hbm,
             srcc, dstc, rows0, rows1, z16, acc_sh,
             gsem0, gsem1, ssem0, ssem1):
    cid = lax.axis_index("c")
    sid = lax.axis_index("s")
    wid = sid * 2 + cid

    _zero2d(z16, 128, 16)
    z0 = sid * 640
    for k in range(5):
        pltpu.sync_copy(z16, acc_sh.at[pl.ds(z0 + k * 128, 128)])
    plsc.subcore_barrier()

    _agg_loop(h2_hbm, src_hbm, dst_hbm, acc_sh, srcc, dstc, (rows0, rows1),
              (rows0, rows1), (gsem0, gsem1), (ssem0, ssem1), wid * BPW2,
              CHUNK2, NCHUNK2, None)

    plsc.subcore_barrier()
    _dump_core_slice(acc_sh, agg_hbm.at[cid], sid)


def _k4_body(h1p_ref, agg_ref, deg_ref, w2_ref, b2_ref, h2_ref):
    norm = deg_ref[0, :, 0:1] + deg_ref[1, :, 0:1] + 1.0
    h1 = jnp.maximum((h1p_ref[...] + agg_ref[0] + agg_ref[1]) / norm, 0.0)
    h2_ref[...] = (
        jnp.dot(h1, w2_ref[...], preferred_element_type=_f32) + b2_ref[...]
    )


def _k6_body(h2_ref, agg_ref, deg_ref, m_ref,
             w1_ref, b1_ref, w2_ref, b2_ref, w3_ref, b3_ref, out_ref):
    norm = deg_ref[0, :, 0:1] + deg_ref[1, :, 0:1] + 1.0
    ge = (h2_ref[...] + agg_ref[0] + agg_ref[1]) / norm
    x = jnp.maximum(jnp.dot(ge, w1_ref[...], preferred_element_type=_f32) + b1_ref[...], 0.0)
    x = jnp.maximum(jnp.dot(x, w2_ref[...], preferred_element_type=_f32) + b2_ref[...], 0.0)
    logits = jnp.dot(x, w3_ref[...], preferred_element_type=_f32) + b3_ref[...]
    sel = jnp.where(m_ref[...] > 0.5, logits, -jnp.inf)
    mx = jnp.max(sel)
    e = jnp.exp(sel - mx)
    out_ref[...] = e / jnp.sum(e)


def kernel(real_features, cat_features, edge_index, mask,
           emb0, emb1, emb2, emb3, W1, b1, W2, b2,
           fc1_w, fc1_b, fc2_w, fc2_b, fc3_w, fc3_b):
    cat = cat_features.astype(_i32)
    src = edge_index[0].astype(_i32)
    dst = edge_index[1].astype(_i32)

    npad = NBLK * EB - E
    pad_src = (jnp.arange(npad, dtype=_i32) * 131) % N
    pad_dst = N + (jnp.arange(npad, dtype=_i32) % 240)
    src_pad = jnp.concatenate([src, pad_src])
    dst_pad = jnp.concatenate([dst, pad_dst])
    src2d = src_pad.reshape(NBLK, EB)
    dst2d = dst_pad.reshape(NBLK, EB)
    src2d_w = src_pad.reshape(NBLK2, EB2)
    dst2d_w = dst_pad.reshape(NBLK2, EB2)

    cat_flat = cat.reshape(NRB, RB, NCAT).transpose(0, 2, 1).reshape(-1)

    W1r = W1[:HID]
    W1e = W1[HID:].reshape(NCAT, 64, HID)
    embs = jnp.stack([emb0, emb1, emb2, emb3])
    b1r = b1.reshape(1, HID)
    b2r = b2.reshape(1, OUT)
    maskf = mask.astype(_f32).reshape(N, 1)

    F = pl.pallas_call(
        _k1_fold,
        grid=(NCAT,),
        in_specs=[
            pl.BlockSpec((1, VOCAB, 64), lambda j: (j, 0, 0)),
            pl.BlockSpec((1, 64, HID), lambda j: (j, 0, 0)),
        ],
        out_specs=pl.BlockSpec((1, VOCAB, HID), lambda j: (j, 0, 0)),
        out_shape=jax.ShapeDtypeStruct((NCAT, VOCAB, HID), _f32),
    )(embs, W1e)

    RBLK = 1000
    R = pl.pallas_call(
        _k1_dense,
        grid=(N // RBLK,),
        in_specs=[
            pl.BlockSpec((RBLK, HID), lambda i: (i, 0)),
            pl.BlockSpec((HID, HID), lambda i: (0, 0)),
            pl.BlockSpec((1, HID), lambda i: (0, 0)),
        ],
        out_specs=pl.BlockSpec((RBLK, HID), lambda i: (i, 0)),
        out_shape=jax.ShapeDtypeStruct((N, HID), _f32),
    )(real_features, W1r, b1r)

    h1p, h1pbf, deg = pl.kernel(
        _k2_body,
        out_type=(
            jax.ShapeDtypeStruct((N, HID), _f32),
            jax.ShapeDtypeStruct((N, HID), _bf16),
            jax.ShapeDtypeStruct((2, N, 16), _f32),
        ),
        mesh=_mesh,
        compiler_params=pltpu.CompilerParams(use_tc_tiling_on_sc=False, needs_layout_passes=False),
        scratch_types=[
            pltpu.VMEM((NCAT * RB,), _i32),
            pltpu.VMEM((NCAT * RB,), _i32),
            pltpu.VMEM((RB, HID), _f32),
            pltpu.VMEM((RB, HID), _f32),
            pltpu.VMEM((RB, HID), _bf16),
            pltpu.VMEM((RB, HID), _bf16),
            pltpu.VMEM((CHUNK2, EB2), _i32),
            pltpu.VMEM((EB2, 16), _f32),
            pltpu.VMEM((128, 16), _f32),
            pltpu.VMEM_SHARED((PADN, 16), _f32),
            pltpu.SemaphoreType.DMA,
            pltpu.SemaphoreType.DMA,
            pltpu.SemaphoreType.DMA,
            pltpu.SemaphoreType.DMA,
            pltpu.SemaphoreType.DMA,
            pltpu.SemaphoreType.DMA,
            pltpu.SemaphoreType.DMA,
            pltpu.SemaphoreType.DMA,
        ],
    )(R, F, cat_flat, dst2d_w)

    agg1 = pl.kernel(
        _k3_body,
        out_type=jax.ShapeDtypeStruct((2, N, HID), _f32),
        mesh=_mesh,
        compiler_params=pltpu.CompilerParams(use_tc_tiling_on_sc=False, needs_layout_passes=False),
        scratch_types=[
            pltpu.VMEM((CHUNK, EB), _i32),
            pltpu.VMEM((CHUNK, EB), _i32),
            pltpu.VMEM((EB, HID), _bf16),
            pltpu.VMEM((EB, HID), _bf16),
            pltpu.VMEM((EB, HID), _f32),
            pltpu.VMEM((EB, HID), _f32),
            pltpu.VMEM_SHARED((PADN, HID), _f32),
            pltpu.SemaphoreType.DMA,
            pltpu.SemaphoreType.DMA,
            pltpu.SemaphoreType.DMA,
            pltpu.SemaphoreType.DMA,
        ],
    )(h1pbf, src2d, dst2d)

    h2 = pl.pallas_call(
        _k4_body,
        grid=(N // RBLK,),
        in_specs=[
            pl.BlockSpec((RBLK, HID), lambda i: (i, 0)),
            pl.BlockSpec((2, RBLK, HID), lambda i: (0, i, 0)),
            pl.BlockSpec((2, RBLK, 16), lambda i: (0, i, 0)),
            pl.BlockSpec((HID, OUT), lambda i: (0, 0)),
            pl.BlockSpec((1, OUT), lambda i: (0, 0)),
        ],
        out_specs=pl.BlockSpec((RBLK, OUT), lambda i: (i, 0)),
        out_shape=jax.ShapeDtypeStruct((N, OUT), _f32),
    )(h1p, agg1, deg, W2, b2r)

    agg2 = pl.kernel(
        _k5_body,
        out_type=jax.ShapeDtypeStruct((2, N, OUT), _f32),
        mesh=_mesh,
        compiler_params=pltpu.CompilerParams(use_tc_tiling_on_sc=False, needs_layout_passes=False),
        scratch_types=[
            pltpu.VMEM((CHUNK2, EB2), _i32),
            pltpu.VMEM((CHUNK2, EB2), _i32),
            pltpu.VMEM((EB2, OUT), _f32),
            pltpu.VMEM((EB2, OUT), _f32),
            pltpu.VMEM((128, 16), _f32),
            pltpu.VMEM_SHARED((PADN, OUT), _f32),
            pltpu.SemaphoreType.DMA,
            pltpu.SemaphoreType.DMA,
            pltpu.SemaphoreType.DMA,
            pltpu.SemaphoreType.DMA,
        ],
    )(h2, src2d_w, dst2d_w)

    probs = pl.pallas_call(
        _k6_body,
        out_shape=jax.ShapeDtypeStruct((N, 1), _f32),
    )(h2, agg2, deg, maskf,
      fc1_w, fc1_b.reshape(1, 24), fc2_w, fc2_b.reshape(1, 24),
      fc3_w, fc3_b.reshape(1, 1))

    return probs.reshape(-1)


# speedup vs baseline: 1.4958x; 1.4958x over previous
import jax
import jax.numpy as jnp
from jax import lax
from jax.experimental import pallas as pl
from jax.experimental.pallas import tpu as pltpu
from jax.experimental.pallas import tpu_sc as plsc

N = 10000
E = 320000
NCAT = 4
VOCAB = 1000
HID = 128
OUT = 16

NW = 32
PADN = 10240

EB = 80
NBLK = 4000
BPW = NBLK // NW
CHUNK = 25
NCHUNK = BPW // CHUNK

EB2 = 512
NBLK2 = 640
BPW2 = NBLK2 // NW
CHUNK2 = 20
NCHUNK2 = BPW2 // CHUNK2

_mesh = plsc.VectorSubcoreMesh(core_axis_name="c", subcore_axis_name="s")
_f32 = jnp.float32
_i32 = jnp.int32


def _zero2d(ref, nrows, width):
    @pl.loop(0, nrows)
    def _(r):
        for c in range(width // 16):
            ref[r, pl.ds(c * 16, 16)] = jnp.zeros((16,), _f32)


def _dump_core_slice(sh, hbm_core, sid):
    d0 = sid * 624
    pltpu.sync_copy(sh.at[pl.ds(d0, 624)], hbm_core.at[pl.ds(d0, 624)])

    @pl.when(sid == 15)
    def _():
        pltpu.sync_copy(sh.at[pl.ds(9984, 16)], hbm_core.at[pl.ds(9984, 16)])


def _agg_loop(tbl_hbm, src_hbm, dst_hbm, acc_sh, srcc, dstc, rows, gsems,
              ssems, start, chunk, nchunk):
    @pl.loop(0, nchunk)
    def _(ci):
        c0 = start + ci * chunk
        pltpu.sync_copy(src_hbm.at[pl.ds(c0, chunk)], srcc)
        pltpu.sync_copy(dst_hbm.at[pl.ds(c0, chunk)], dstc)
        pltpu.async_copy(tbl_hbm.at[srcc.at[0]], rows[0], gsems[0])

        @pl.loop(0, chunk // 2)
        def _(pi):
            for half in range(2):
                b = pi * 2 + half
                o = 1 - half

                @pl.when(b >= 1)
                def _():
                    pltpu.make_async_copy(
                        rows[o], acc_sh.at[dstc.at[0]], ssems[o]
                    ).wait()

                @pl.when(b + 1 < chunk)
                def _():
                    pltpu.async_copy(tbl_hbm.at[srcc.at[b + 1]], rows[o], gsems[o])

                pltpu.make_async_copy(
                    tbl_hbm.at[srcc.at[b]], rows[half], gsems[half]
                ).wait()
                pltpu.async_copy(
                    rows[half], acc_sh.at[dstc.at[b]], ssems[half], add=True
                )

        if chunk % 2:
            bl = chunk - 1
            pltpu.make_async_copy(rows[1], acc_sh.at[dstc.at[0]], ssems[1]).wait()
            pltpu.make_async_copy(tbl_hbm.at[srcc.at[bl]], rows[0], gsems[0]).wait()
            pltpu.async_copy(rows[0], acc_sh.at[dstc.at[bl]], ssems[0], add=True)
            pltpu.make_async_copy(rows[0], acc_sh.at[dstc.at[0]], ssems[0]).wait()
        else:
            pltpu.make_async_copy(rows[1], acc_sh.at[dstc.at[0]], ssems[1]).wait()


def _k1_fold(emb_ref, w_ref, f_ref):
    f_ref[0] = jnp.dot(emb_ref[0], w_ref[0], preferred_element_type=_f32)


def _k1_dense(x_ref, w_ref, b_ref, r_ref):
    r_ref[...] = (
        jnp.dot(x_ref[...], w_ref[...], preferred_element_type=_f32) + b_ref[...]
    )


RB = 80
NRB = N // RB


def _k2_body(r_hbm, f_hbm, cat_hbm, dst_hbm, h1p_hbm, deg_hbm,
             idx0, idx1, acc0, acc1, dstc, ones_v, z16, deg_sh,
             gsem0, gsem1, wsem0, wsem1, ssem0, ssem1):
    cid = lax.axis_index("c")
    sid = lax.axis_index("s")
    wid = sid * 2 + cid
    idxs = (idx0, idx1)
    accs = (acc0, acc1)
    gsems = (gsem0, gsem1)
    wsems = (wsem0, wsem1)
    ssems = (ssem0, ssem1)

    @pl.loop(0, EB2)
    def _(r):
        ones_v[r, :] = jnp.ones((16,), _f32)

    _zero2d(z16, 128, 16)
    z0 = sid * 640
    for k in range(5):
        pltpu.sync_copy(z16, deg_sh.at[pl.ds(z0 + k * 128, 128)])
    plsc.subcore_barrier()

    nblk = jnp.where(wid < 29, 4, 3)
    start = jnp.where(wid < 29, 4 * wid, 3 * wid + 29)

    for bi in range(5):
        k = bi % 2
        o = (bi - 1) % 2

        if bi < 4:
            @pl.when(bi < nblk)
            def _():
                b = start + bi
                if bi >= 2:
                    pltpu.make_async_copy(
                        accs[k], h1p_hbm.at[pl.ds(0, RB)], wsems[k]
                    ).wait()
                pltpu.sync_copy(
                    cat_hbm.at[pl.ds(b * (NCAT * RB), NCAT * RB)], idxs[k]
                )
                pltpu.sync_copy(r_hbm.at[pl.ds(b * RB, RB)], accs[k])
                for j in range(NCAT):
                    pltpu.async_copy(
                        f_hbm.at[j].at[idxs[k].at[pl.ds(j * RB, RB)]],
                        accs[k], gsems[k], add=True,
                    )

        if bi >= 1:
            @pl.when(bi - 1 < nblk)
            def _():
                bp = start + (bi - 1)
                for j in range(NCAT):
                    pltpu.make_async_copy(
                        f_hbm.at[j].at[idxs[o].at[pl.ds(j * RB, RB)]],
                        accs[o], gsems[o],
                    ).wait()
                pltpu.async_copy(
                    accs[o], h1p_hbm.at[pl.ds(bp * RB, RB)], wsems[o]
                )

    estart = wid * BPW2

    @pl.loop(0, NCHUNK2)
    def _(ci):
        pltpu.sync_copy(dst_hbm.at[pl.ds(estart + ci * CHUNK2, CHUNK2)], dstc)

        @pl.loop(0, CHUNK2 // 2)
        def _(pi):
            for half in range(2):
                b = pi * 2 + half

                @pl.when(b >= 2)
                def _():
                    pltpu.make_async_copy(
                        ones_v, deg_sh.at[dstc.at[0]], ssems[half]
                    ).wait()

                pltpu.async_copy(
                    ones_v, deg_sh.at[dstc.at[b]], ssems[half], add=True
                )

        pltpu.make_async_copy(ones_v, deg_sh.at[dstc.at[0]], ssems[0]).wait()
        pltpu.make_async_copy(ones_v, deg_sh.at[dstc.at[0]], ssems[1]).wait()

    pltpu.make_async_copy(acc0, h1p_hbm.at[pl.ds(0, RB)], wsem0).wait()
    pltpu.make_async_copy(acc1, h1p_hbm.at[pl.ds(0, RB)], wsem1).wait()

    plsc.subcore_barrier()
    _dump_core_slice(deg_sh, deg_hbm.at[cid], sid)


def _k3_body(h1p_hbm, src_hbm, dst_hbm, agg_hbm,
             srcc, dstc, rows0, rows1, acc_sh, gsem0, gsem1, ssem0, ssem1):
    cid = lax.axis_index("c")
    sid = lax.axis_index("s")
    wid = sid * 2 + cid

    _zero2d(rows0, EB, HID)
    z0 = sid * 640
    for k in range(640 // EB):
        pltpu.sync_copy(rows0, acc_sh.at[pl.ds(z0 + k * EB, EB)])
    plsc.subcore_barrier()

    _agg_loop(h1p_hbm, src_hbm, dst_hbm, acc_sh, srcc, dstc, (rows0, rows1),
              (gsem0, gsem1), (ssem0, ssem1), wid * BPW, CHUNK, NCHUNK)

    plsc.subcore_barrier()
    _dump_core_slice(acc_sh, agg_hbm.at[cid], sid)


def _k5_body(h2_hbm, src_hbm, dst_hbm, agg_hbm,
             srcc, dstc, rows0, rows1, z16, acc_sh,
             gsem0, gsem1, ssem0, ssem1):
    cid = lax.axis_index("c")
    sid = lax.axis_index("s")
    wid = sid * 2 + cid

    _zero2d(z16, 128, 16)
    z0 = sid * 640
    for k in range(5):
        pltpu.sync_copy(z16, acc_sh.at[pl.ds(z0 + k * 128, 128)])
    plsc.subcore_barrier()

    _agg_loop(h2_hbm, src_hbm, dst_hbm, acc_sh, srcc, dstc, (rows0, rows1),
              (gsem0, gsem1), (ssem0, ssem1), wid * BPW2, CHUNK2, NCHUNK2)

    plsc.subcore_barrier()
    _dump_core_slice(acc_sh, agg_hbm.at[cid], sid)


def _k4_body(h1p_ref, agg_ref, deg_ref, w2_ref, b2_ref, h2_ref):
    norm = deg_ref[0, :, 0:1] + deg_ref[1, :, 0:1] + 1.0
    h1 = jnp.maximum((h1p_ref[...] + agg_ref[0] + agg_ref[1]) / norm, 0.0)
    h2_ref[...] = (
        jnp.dot(h1, w2_ref[...], preferred_element_type=_f32) + b2_ref[...]
    )


def _k6_body(h2_ref, agg_ref, deg_ref, m_ref,
             w1_ref, b1_ref, w2_ref, b2_ref, w3_ref, b3_ref, out_ref):
    norm = deg_ref[0, :, 0:1] + deg_ref[1, :, 0:1] + 1.0
    ge = (h2_ref[...] + agg_ref[0] + agg_ref[1]) / norm
    x = jnp.maximum(jnp.dot(ge, w1_ref[...], preferred_element_type=_f32) + b1_ref[...], 0.0)
    x = jnp.maximum(jnp.dot(x, w2_ref[...], preferred_element_type=_f32) + b2_ref[...], 0.0)
    logits = jnp.dot(x, w3_ref[...], preferred_element_type=_f32) + b3_ref[...]
    sel = jnp.where(m_ref[...] > 0.5, logits, -jnp.inf)
    mx = jnp.max(sel)
    e = jnp.exp(sel - mx)
    out_ref[...] = e / jnp.sum(e)


def kernel(real_features, cat_features, edge_index, mask,
           emb0, emb1, emb2, emb3, W1, b1, W2, b2,
           fc1_w, fc1_b, fc2_w, fc2_b, fc3_w, fc3_b):
    cat = cat_features.astype(_i32)
    src = edge_index[0].astype(_i32)
    dst = edge_index[1].astype(_i32)

    src2d = src.reshape(NBLK, EB)
    dst2d = dst.reshape(NBLK, EB)
    npad = NBLK2 * EB2 - E
    pad_src = (jnp.arange(npad, dtype=_i32) * 131) % N
    pad_dst = N + (jnp.arange(npad, dtype=_i32) % 240)
    src2d_w = jnp.concatenate([src, pad_src]).reshape(NBLK2, EB2)
    dst2d_w = jnp.concatenate([dst, pad_dst]).reshape(NBLK2, EB2)

    cat_flat = cat.reshape(NRB, RB, NCAT).transpose(0, 2, 1).reshape(-1)

    W1r = W1[:HID]
    W1e = W1[HID:].reshape(NCAT, 64, HID)
    embs = jnp.stack([emb0, emb1, emb2, emb3])
    b1r = b1.reshape(1, HID)
    b2r = b2.reshape(1, OUT)
    maskf = mask.astype(_f32).reshape(N, 1)

    F = pl.pallas_call(
        _k1_fold,
        grid=(NCAT,),
        in_specs=[
            pl.BlockSpec((1, VOCAB, 64), lambda j: (j, 0, 0)),
            pl.BlockSpec((1, 64, HID), lambda j: (j, 0, 0)),
        ],
        out_specs=pl.BlockSpec((1, VOCAB, HID), lambda j: (j, 0, 0)),
        out_shape=jax.ShapeDtypeStruct((NCAT, VOCAB, HID), _f32),
    )(embs, W1e)

    RBLK = 1000
    R = pl.pallas_call(
        _k1_dense,
        grid=(N // RBLK,),
        in_specs=[
            pl.BlockSpec((RBLK, HID), lambda i: (i, 0)),
            pl.BlockSpec((HID, HID), lambda i: (0, 0)),
            pl.BlockSpec((1, HID), lambda i: (0, 0)),
        ],
        out_specs=pl.BlockSpec((RBLK, HID), lambda i: (i, 0)),
        out_shape=jax.ShapeDtypeStruct((N, HID), _f32),
    )(real_features, W1r, b1r)

    h1p, deg = pl.kernel(
        _k2_body,
        out_type=(
            jax.ShapeDtypeStruct((N, HID), _f32),
            jax.ShapeDtypeStruct((2, N, 16), _f32),
        ),
        mesh=_mesh,
        compiler_params=pltpu.CompilerParams(use_tc_tiling_on_sc=False),
        scratch_types=[
            pltpu.VMEM((NCAT * RB,), _i32),
            pltpu.VMEM((NCAT * RB,), _i32),
            pltpu.VMEM((RB, HID), _f32),
            pltpu.VMEM((RB, HID), _f32),
            pltpu.VMEM((CHUNK2, EB2), _i32),
            pltpu.VMEM((EB2, 16), _f32),
            pltpu.VMEM((128, 16), _f32),
            pltpu.VMEM_SHARED((PADN, 16), _f32),
            pltpu.SemaphoreType.DMA,
            pltpu.SemaphoreType.DMA,
            pltpu.SemaphoreType.DMA,
            pltpu.SemaphoreType.DMA,
            pltpu.SemaphoreType.DMA,
            pltpu.SemaphoreType.DMA,
        ],
    )(R, F, cat_flat, dst2d_w)

    agg1 = pl.kernel(
        _k3_body,
        out_type=jax.ShapeDtypeStruct((2, N, HID), _f32),
        mesh=_mesh,
        compiler_params=pltpu.CompilerParams(use_tc_tiling_on_sc=False),
        scratch_types=[
            pltpu.VMEM((CHUNK, EB), _i32),
            pltpu.VMEM((CHUNK, EB), _i32),
            pltpu.VMEM((EB, HID), _f32),
            pltpu.VMEM((EB, HID), _f32),
            pltpu.VMEM_SHARED((PADN, HID), _f32),
            pltpu.SemaphoreType.DMA,
            pltpu.SemaphoreType.DMA,
            pltpu.SemaphoreType.DMA,
            pltpu.SemaphoreType.DMA,
        ],
    )(h1p, src2d, dst2d)

    h2 = pl.pallas_call(
        _k4_body,
        grid=(N // RBLK,),
        in_specs=[
            pl.BlockSpec((RBLK, HID), lambda i: (i, 0)),
            pl.BlockSpec((2, RBLK, HID), lambda i: (0, i, 0)),
            pl.BlockSpec((2, RBLK, 16), lambda i: (0, i, 0)),
            pl.BlockSpec((HID, OUT), lambda i: (0, 0)),
            pl.BlockSpec((1, OUT), lambda i: (0, 0)),
        ],
        out_specs=pl.BlockSpec((RBLK, OUT), lambda i: (i, 0)),
        out_shape=jax.ShapeDtypeStruct((N, OUT), _f32),
    )(h1p, agg1, deg, W2, b2r)

    agg2 = pl.kernel(
        _k5_body,
        out_type=jax.ShapeDtypeStruct((2, N, OUT), _f32),
        mesh=_mesh,
        compiler_params=pltpu.CompilerParams(use_tc_tiling_on_sc=False),
        scratch_types=[
            pltpu.VMEM((CHUNK2, EB2), _i32),
            pltpu.VMEM((CHUNK2, EB2), _i32),
            pltpu.VMEM((EB2, OUT), _f32),
            pltpu.VMEM((EB2, OUT), _f32),
            pltpu.VMEM((128, 16), _f32),
            pltpu.VMEM_SHARED((PADN, OUT), _f32),
            pltpu.SemaphoreType.DMA,
            pltpu.SemaphoreType.DMA,
            pltpu.SemaphoreType.DMA,
            pltpu.SemaphoreType.DMA,
        ],
    )(h2, src2d_w, dst2d_w)

    probs = pl.pallas_call(
        _k6_body,
        out_shape=jax.ShapeDtypeStruct((N, 1), _f32),
    )(h2, agg2, deg, maskf,
      fc1_w, fc1_b.reshape(1, 24), fc2_w, fc2_b.reshape(1, 24),
      fc3_w, fc3_b.reshape(1, 1))

    return probs.reshape(-1)


# speedup vs baseline: 1.5027x; 1.0046x over previous
import jax
import jax.numpy as jnp
from jax import lax
from jax.experimental import pallas as pl
from jax.experimental.pallas import tpu as pltpu
from jax.experimental.pallas import tpu_sc as plsc

N = 10000
E = 320000
NCAT = 4
VOCAB = 1000
HID = 128
OUT = 16

NW = 32
PADN = 10240

EB = 80
NBLK = 4000
BPW = NBLK // NW
CHUNK = 25
NCHUNK = BPW // CHUNK

EB2 = 512
NBLK2 = 640
BPW2 = NBLK2 // NW
CHUNK2 = 20
NCHUNK2 = BPW2 // CHUNK2

_mesh = plsc.VectorSubcoreMesh(core_axis_name="c", subcore_axis_name="s")
_f32 = jnp.float32
_i32 = jnp.int32


def _zero2d(ref, nrows, width):
    @pl.loop(0, nrows)
    def _(r):
        for c in range(width // 16):
            ref[r, pl.ds(c * 16, 16)] = jnp.zeros((16,), _f32)


def _dump_core_slice(sh, hbm_core, sid):
    d0 = sid * 624
    pltpu.sync_copy(sh.at[pl.ds(d0, 624)], hbm_core.at[pl.ds(d0, 624)])

    @pl.when(sid == 15)
    def _():
        pltpu.sync_copy(sh.at[pl.ds(9984, 16)], hbm_core.at[pl.ds(9984, 16)])


def _agg_loop(tbl_hbm, src_hbm, dst_hbm, acc_sh, srcc, dstc, rows, gsems,
              ssems, start, chunk, nchunk):
    @pl.loop(0, nchunk)
    def _(ci):
        c0 = start + ci * chunk
        pltpu.sync_copy(src_hbm.at[pl.ds(c0, chunk)], srcc)
        pltpu.sync_copy(dst_hbm.at[pl.ds(c0, chunk)], dstc)
        pltpu.async_copy(tbl_hbm.at[srcc.at[0]], rows[0], gsems[0])

        @pl.loop(0, chunk // 2)
        def _(pi):
            for half in range(2):
                b = pi * 2 + half
                o = 1 - half

                @pl.when(b >= 1)
                def _():
                    pltpu.make_async_copy(
                        rows[o], acc_sh.at[dstc.at[0]], ssems[o]
                    ).wait()

                @pl.when(b + 1 < chunk)
                def _():
                    pltpu.async_copy(tbl_hbm.at[srcc.at[b + 1]], rows[o], gsems[o])

                pltpu.make_async_copy(
                    tbl_hbm.at[srcc.at[b]], rows[half], gsems[half]
                ).wait()
                pltpu.async_copy(
                    rows[half], acc_sh.at[dstc.at[b]], ssems[half], add=True
                )

        if chunk % 2:
            bl = chunk - 1
            pltpu.make_async_copy(rows[1], acc_sh.at[dstc.at[0]], ssems[1]).wait()
            pltpu.make_async_copy(tbl_hbm.at[srcc.at[bl]], rows[0], gsems[0]).wait()
            pltpu.async_copy(rows[0], acc_sh.at[dstc.at[bl]], ssems[0], add=True)
            pltpu.make_async_copy(rows[0], acc_sh.at[dstc.at[0]], ssems[0]).wait()
        else:
            pltpu.make_async_copy(rows[1], acc_sh.at[dstc.at[0]], ssems[1]).wait()


def _k1_fold(emb_ref, w_ref, f_ref):
    f_ref[0] = jnp.dot(emb_ref[0], w_ref[0], preferred_element_type=_f32)


def _k1_dense(x_ref, w_ref, b_ref, r_ref):
    r_ref[...] = (
        jnp.dot(x_ref[...], w_ref[...], preferred_element_type=_f32) + b_ref[...]
    )


RB = 80
NRB = N // RB


def _k2_body(r_hbm, f_hbm, cat_hbm, dst_hbm, h1p_hbm, deg_hbm,
             idx0, idx1, acc0, acc1, dstc, ones_v, z16, deg_sh,
             gsem0, gsem1, wsem0, wsem1, ssem0, ssem1):
    cid = lax.axis_index("c")
    sid = lax.axis_index("s")
    wid = sid * 2 + cid
    idxs = (idx0, idx1)
    accs = (acc0, acc1)
    gsems = (gsem0, gsem1)
    wsems = (wsem0, wsem1)
    ssems = (ssem0, ssem1)

    @pl.loop(0, EB2)
    def _(r):
        ones_v[r, :] = jnp.ones((16,), _f32)

    _zero2d(z16, 128, 16)
    z0 = sid * 640
    for k in range(5):
        pltpu.sync_copy(z16, deg_sh.at[pl.ds(z0 + k * 128, 128)])
    plsc.subcore_barrier()

    nblk = jnp.where(wid < 29, 4, 3)
    start = jnp.where(wid < 29, 4 * wid, 3 * wid + 29)

    for bi in range(5):
        k = bi % 2
        o = (bi - 1) % 2

        if bi < 4:
            @pl.when(bi < nblk)
            def _():
                b = start + bi
                if bi >= 2:
                    pltpu.make_async_copy(
                        accs[k], h1p_hbm.at[pl.ds(0, RB)], wsems[k]
                    ).wait()
                pltpu.sync_copy(
                    cat_hbm.at[pl.ds(b * (NCAT * RB), NCAT * RB)], idxs[k]
                )
                pltpu.sync_copy(r_hbm.at[pl.ds(b * RB, RB)], accs[k])
                for j in range(NCAT):
                    pltpu.async_copy(
                        f_hbm.at[j].at[idxs[k].at[pl.ds(j * RB, RB)]],
                        accs[k], gsems[k], add=True,
                    )

        if bi >= 1:
            @pl.when(bi - 1 < nblk)
            def _():
                bp = start + (bi - 1)
                for j in range(NCAT):
                    pltpu.make_async_copy(
                        f_hbm.at[j].at[idxs[o].at[pl.ds(j * RB, RB)]],
                        accs[o], gsems[o],
                    ).wait()
                pltpu.async_copy(
                    accs[o], h1p_hbm.at[pl.ds(bp * RB, RB)], wsems[o]
                )

    estart = wid * BPW2

    @pl.loop(0, NCHUNK2)
    def _(ci):
        pltpu.sync_copy(dst_hbm.at[pl.ds(estart + ci * CHUNK2, CHUNK2)], dstc)

        @pl.loop(0, CHUNK2 // 2)
        def _(pi):
            for half in range(2):
                b = pi * 2 + half

                @pl.when(b >= 2)
                def _():
                    pltpu.make_async_copy(
                        ones_v, deg_sh.at[dstc.at[0]], ssems[half]
                    ).wait()

                pltpu.async_copy(
                    ones_v, deg_sh.at[dstc.at[b]], ssems[half], add=True
                )

        pltpu.make_async_copy(ones_v, deg_sh.at[dstc.at[0]], ssems[0]).wait()
        pltpu.make_async_copy(ones_v, deg_sh.at[dstc.at[0]], ssems[1]).wait()

    pltpu.make_async_copy(acc0, h1p_hbm.at[pl.ds(0, RB)], wsem0).wait()
    pltpu.make_async_copy(acc1, h1p_hbm.at[pl.ds(0, RB)], wsem1).wait()

    plsc.subcore_barrier()
    _dump_core_slice(deg_sh, deg_hbm.at[cid], sid)


def _k3_body(h1p_hbm, src_hbm, dst_hbm, agg_hbm,
             srcc, dstc, rows0, rows1, acc_sh, gsem0, gsem1, ssem0, ssem1):
    cid = lax.axis_index("c")
    sid = lax.axis_index("s")
    wid = sid * 2 + cid

    _zero2d(rows0, EB, HID)
    z0 = sid * 640
    for k in range(640 // EB):
        pltpu.sync_copy(rows0, acc_sh.at[pl.ds(z0 + k * EB, EB)])
    plsc.subcore_barrier()

    _agg_loop(h1p_hbm, src_hbm, dst_hbm, acc_sh, srcc, dstc, (rows0, rows1),
              (gsem0, gsem1), (ssem0, ssem1), wid * BPW, CHUNK, NCHUNK)

    plsc.subcore_barrier()
    _dump_core_slice(acc_sh, agg_hbm.at[cid], sid)


def _k5_body(h2_hbm, src_hbm, dst_hbm, agg_hbm,
             srcc, dstc, rows0, rows1, z16, h2_sh, acc_sh,
             gsem0, gsem1, ssem0, ssem1):
    cid = lax.axis_index("c")
    sid = lax.axis_index("s")
    wid = sid * 2 + cid

    _zero2d(z16, 128, 16)
    z0 = sid * 640
    for k in range(5):
        pltpu.sync_copy(z16, acc_sh.at[pl.ds(z0 + k * 128, 128)])
    pltpu.sync_copy(h2_hbm.at[pl.ds(sid * 625, 625)], h2_sh.at[pl.ds(sid * 625, 625)])
    plsc.subcore_barrier()

    _agg_loop(h2_sh, src_hbm, dst_hbm, acc_sh, srcc, dstc, (rows0, rows1),
              (gsem0, gsem1), (ssem0, ssem1), wid * BPW2, CHUNK2, NCHUNK2)

    plsc.subcore_barrier()
    _dump_core_slice(acc_sh, agg_hbm.at[cid], sid)


def _k4_body(h1p_ref, agg_ref, deg_ref, w2_ref, b2_ref, h2_ref):
    norm = deg_ref[0, :, 0:1] + deg_ref[1, :, 0:1] + 1.0
    h1 = jnp.maximum((h1p_ref[...] + agg_ref[0] + agg_ref[1]) / norm, 0.0)
    h2_ref[...] = (
        jnp.dot(h1, w2_ref[...], preferred_element_type=_f32) + b2_ref[...]
    )


def _k6_body(h2_ref, agg_ref, deg_ref, m_ref,
             w1_ref, b1_ref, w2_ref, b2_ref, w3_ref, b3_ref, out_ref):
    norm = deg_ref[0, :, 0:1] + deg_ref[1, :, 0:1] + 1.0
    ge = (h2_ref[...] + agg_ref[0] + agg_ref[1]) / norm
    x = jnp.maximum(jnp.dot(ge, w1_ref[...], preferred_element_type=_f32) + b1_ref[...], 0.0)
    x = jnp.maximum(jnp.dot(x, w2_ref[...], preferred_element_type=_f32) + b2_ref[...], 0.0)
    logits = jnp.dot(x, w3_ref[...], preferred_element_type=_f32) + b3_ref[...]
    sel = jnp.where(m_ref[...] > 0.5, logits, -jnp.inf)
    mx = jnp.max(sel)
    e = jnp.exp(sel - mx)
    out_ref[...] = e / jnp.sum(e)


def kernel(real_features, cat_features, edge_index, mask,
           emb0, emb1, emb2, emb3, W1, b1, W2, b2,
           fc1_w, fc1_b, fc2_w, fc2_b, fc3_w, fc3_b):
    cat = cat_features.astype(_i32)
    src = edge_index[0].astype(_i32)
    dst = edge_index[1].astype(_i32)

    src2d = src.reshape(NBLK, EB)
    dst2d = dst.reshape(NBLK, EB)
    npad = NBLK2 * EB2 - E
    pad_src = (jnp.arange(npad, dtype=_i32) * 131) % N
    pad_dst = N + (jnp.arange(npad, dtype=_i32) % 240)
    src2d_w = jnp.concatenate([src, pad_src]).reshape(NBLK2, EB2)
    dst2d_w = jnp.concatenate([dst, pad_dst]).reshape(NBLK2, EB2)

    cat_flat = cat.reshape(NRB, RB, NCAT).transpose(0, 2, 1).reshape(-1)

    W1r = W1[:HID]
    W1e = W1[HID:].reshape(NCAT, 64, HID)
    embs = jnp.stack([emb0, emb1, emb2, emb3])
    b1r = b1.reshape(1, HID)
    b2r = b2.reshape(1, OUT)
    maskf = mask.astype(_f32).reshape(N, 1)

    F = pl.pallas_call(
        _k1_fold,
        grid=(NCAT,),
        in_specs=[
            pl.BlockSpec((1, VOCAB, 64), lambda j: (j, 0, 0)),
            pl.BlockSpec((1, 64, HID), lambda j: (j, 0, 0)),
        ],
        out_specs=pl.BlockSpec((1, VOCAB, HID), lambda j: (j, 0, 0)),
        out_shape=jax.ShapeDtypeStruct((NCAT, VOCAB, HID), _f32),
    )(embs, W1e)

    RBLK = 1000
    R = pl.pallas_call(
        _k1_dense,
        grid=(N // RBLK,),
        in_specs=[
            pl.BlockSpec((RBLK, HID), lambda i: (i, 0)),
            pl.BlockSpec((HID, HID), lambda i: (0, 0)),
            pl.BlockSpec((1, HID), lambda i: (0, 0)),
        ],
        out_specs=pl.BlockSpec((RBLK, HID), lambda i: (i, 0)),
        out_shape=jax.ShapeDtypeStruct((N, HID), _f32),
    )(real_features, W1r, b1r)

    h1p, deg = pl.kernel(
        _k2_body,
        out_type=(
            jax.ShapeDtypeStruct((N, HID), _f32),
            jax.ShapeDtypeStruct((2, N, 16), _f32),
        ),
        mesh=_mesh,
        compiler_params=pltpu.CompilerParams(use_tc_tiling_on_sc=False),
        scratch_types=[
            pltpu.VMEM((NCAT * RB,), _i32),
            pltpu.VMEM((NCAT * RB,), _i32),
            pltpu.VMEM((RB, HID), _f32),
            pltpu.VMEM((RB, HID), _f32),
            pltpu.VMEM((CHUNK2, EB2), _i32),
            pltpu.VMEM((EB2, 16), _f32),
            pltpu.VMEM((128, 16), _f32),
            pltpu.VMEM_SHARED((PADN, 16), _f32),
            pltpu.SemaphoreType.DMA,
            pltpu.SemaphoreType.DMA,
            pltpu.SemaphoreType.DMA,
            pltpu.SemaphoreType.DMA,
            pltpu.SemaphoreType.DMA,
            pltpu.SemaphoreType.DMA,
        ],
    )(R, F, cat_flat, dst2d_w)

    agg1 = pl.kernel(
        _k3_body,
        out_type=jax.ShapeDtypeStruct((2, N, HID), _f32),
        mesh=_mesh,
        compiler_params=pltpu.CompilerParams(use_tc_tiling_on_sc=False),
        scratch_types=[
            pltpu.VMEM((CHUNK, EB), _i32),
            pltpu.VMEM((CHUNK, EB), _i32),
            pltpu.VMEM((EB, HID), _f32),
            pltpu.VMEM((EB, HID), _f32),
            pltpu.VMEM_SHARED((PADN, HID), _f32),
            pltpu.SemaphoreType.DMA,
            pltpu.SemaphoreType.DMA,
            pltpu.SemaphoreType.DMA,
            pltpu.SemaphoreType.DMA,
        ],
    )(h1p, src2d, dst2d)

    h2 = pl.pallas_call(
        _k4_body,
        grid=(N // RBLK,),
        in_specs=[
            pl.BlockSpec((RBLK, HID), lambda i: (i, 0)),
            pl.BlockSpec((2, RBLK, HID), lambda i: (0, i, 0)),
            pl.BlockSpec((2, RBLK, 16), lambda i: (0, i, 0)),
            pl.BlockSpec((HID, OUT), lambda i: (0, 0)),
            pl.BlockSpec((1, OUT), lambda i: (0, 0)),
        ],
        out_specs=pl.BlockSpec((RBLK, OUT), lambda i: (i, 0)),
        out_shape=jax.ShapeDtypeStruct((N, OUT), _f32),
    )(h1p, agg1, deg, W2, b2r)

    agg2 = pl.kernel(
        _k5_body,
        out_type=jax.ShapeDtypeStruct((2, N, OUT), _f32),
        mesh=_mesh,
        compiler_params=pltpu.CompilerParams(use_tc_tiling_on_sc=False),
        scratch_types=[
            pltpu.VMEM((CHUNK2, EB2), _i32),
            pltpu.VMEM((CHUNK2, EB2), _i32),
            pltpu.VMEM((EB2, OUT), _f32),
            pltpu.VMEM((EB2, OUT), _f32),
            pltpu.VMEM((128, 16), _f32),
            pltpu.VMEM_SHARED((N, OUT), _f32),
            pltpu.VMEM_SHARED((PADN, OUT), _f32),
            pltpu.SemaphoreType.DMA,
            pltpu.SemaphoreType.DMA,
            pltpu.SemaphoreType.DMA,
            pltpu.SemaphoreType.DMA,
        ],
    )(h2, src2d_w, dst2d_w)

    probs = pl.pallas_call(
        _k6_body,
        out_shape=jax.ShapeDtypeStruct((N, 1), _f32),
    )(h2, agg2, deg, maskf,
      fc1_w, fc1_b.reshape(1, 24), fc2_w, fc2_b.reshape(1, 24),
      fc3_w, fc3_b.reshape(1, 1))

    return probs.reshape(-1)


# speedup vs baseline: 1.5066x; 1.0026x over previous
import jax
import jax.numpy as jnp
from jax import lax
from jax.experimental import pallas as pl
from jax.experimental.pallas import tpu as pltpu
from jax.experimental.pallas import tpu_sc as plsc

N = 10000
E = 320000
NCAT = 4
VOCAB = 1000
HID = 128
OUT = 16

NW = 32
PADN = 10240

EB = 80
NBLK = 4000
BPW = NBLK // NW
CHUNK = 25
NCHUNK = BPW // CHUNK

EB2 = 512
NBLK2 = 640
BPW2 = NBLK2 // NW
CHUNK2 = 20
NCHUNK2 = BPW2 // CHUNK2

_mesh = plsc.VectorSubcoreMesh(core_axis_name="c", subcore_axis_name="s")
_f32 = jnp.float32
_i32 = jnp.int32


def _zero2d(ref, nrows, width):
    @pl.loop(0, nrows)
    def _(r):
        for c in range(width // 16):
            ref[r, pl.ds(c * 16, 16)] = jnp.zeros((16,), _f32)


def _dump_core_slice(sh, hbm_core, sid):
    d0 = sid * 624
    pltpu.sync_copy(sh.at[pl.ds(d0, 624)], hbm_core.at[pl.ds(d0, 624)])

    @pl.when(sid == 15)
    def _():
        pltpu.sync_copy(sh.at[pl.ds(9984, 16)], hbm_core.at[pl.ds(9984, 16)])


def _agg_loop(tbl_hbm, src_hbm, dst_hbm, acc_sh, srcc, dstc, rows, gsems,
              ssems, start, chunk, nchunk):
    @pl.loop(0, nchunk)
    def _(ci):
        c0 = start + ci * chunk
        pltpu.sync_copy(src_hbm.at[pl.ds(c0, chunk)], srcc)
        pltpu.sync_copy(dst_hbm.at[pl.ds(c0, chunk)], dstc)
        pltpu.async_copy(tbl_hbm.at[srcc.at[0]], rows[0], gsems[0])

        @pl.loop(0, chunk // 2)
        def _(pi):
            for half in range(2):
                b = pi * 2 + half
                o = 1 - half

                @pl.when(b >= 1)
                def _():
                    pltpu.make_async_copy(
                        rows[o], acc_sh.at[dstc.at[0]], ssems[o]
                    ).wait()

                @pl.when(b + 1 < chunk)
                def _():
                    pltpu.async_copy(tbl_hbm.at[srcc.at[b + 1]], rows[o], gsems[o])

                pltpu.make_async_copy(
                    tbl_hbm.at[srcc.at[b]], rows[half], gsems[half]
                ).wait()
                pltpu.async_copy(
                    rows[half], acc_sh.at[dstc.at[b]], ssems[half], add=True
                )

        if chunk % 2:
            bl = chunk - 1
            pltpu.make_async_copy(rows[1], acc_sh.at[dstc.at[0]], ssems[1]).wait()
            pltpu.make_async_copy(tbl_hbm.at[srcc.at[bl]], rows[0], gsems[0]).wait()
            pltpu.async_copy(rows[0], acc_sh.at[dstc.at[bl]], ssems[0], add=True)
            pltpu.make_async_copy(rows[0], acc_sh.at[dstc.at[0]], ssems[0]).wait()
        else:
            pltpu.make_async_copy(rows[1], acc_sh.at[dstc.at[0]], ssems[1]).wait()


def _k1_fold(emb_ref, w_ref, f_ref):
    f_ref[0] = jnp.dot(emb_ref[0], w_ref[0], preferred_element_type=_f32)


def _k1_dense(x_ref, w_ref, b_ref, r_ref):
    r_ref[...] = (
        jnp.dot(x_ref[...], w_ref[...], preferred_element_type=_f32) + b_ref[...]
    )


RB = 80
NRB = N // RB


def _k2_body(r_hbm, f_hbm, cat_hbm, dst_hbm, h1p_hbm, deg_hbm,
             idx0, idx1, acc0, acc1, dstc, ones_v, z16, f_sh, deg_sh,
             gsem0, gsem1, wsem0, wsem1, ssem0, ssem1):
    cid = lax.axis_index("c")
    sid = lax.axis_index("s")
    wid = sid * 2 + cid
    idxs = (idx0, idx1)
    accs = (acc0, acc1)
    gsems = (gsem0, gsem1)
    wsems = (wsem0, wsem1)
    ssems = (ssem0, ssem1)

    @pl.loop(0, EB2)
    def _(r):
        ones_v[r, :] = jnp.ones((16,), _f32)

    _zero2d(z16, 128, 16)
    z0 = sid * 640
    for k in range(5):
        pltpu.sync_copy(z16, deg_sh.at[pl.ds(z0 + k * 128, 128)])
    pltpu.sync_copy(f_hbm.at[pl.ds(sid * 250, 250)], f_sh.at[pl.ds(sid * 250, 250)])
    plsc.subcore_barrier()

    nblk = jnp.where(wid < 29, 4, 3)
    start = jnp.where(wid < 29, 4 * wid, 3 * wid + 29)

    for bi in range(5):
        k = bi % 2
        o = (bi - 1) % 2

        if bi < 4:
            @pl.when(bi < nblk)
            def _():
                b = start + bi
                if bi >= 2:
                    pltpu.make_async_copy(
                        accs[k], h1p_hbm.at[pl.ds(0, RB)], wsems[k]
                    ).wait()
                pltpu.sync_copy(
                    cat_hbm.at[pl.ds(b * (NCAT * RB), NCAT * RB)], idxs[k]
                )
                pltpu.sync_copy(r_hbm.at[pl.ds(b * RB, RB)], accs[k])
                for j in range(NCAT):
                    pltpu.async_copy(
                        f_sh.at[idxs[k].at[pl.ds(j * RB, RB)]],
                        accs[k], gsems[k], add=True,
                    )

        if bi >= 1:
            @pl.when(bi - 1 < nblk)
            def _():
                bp = start + (bi - 1)
                for j in range(NCAT):
                    pltpu.make_async_copy(
                        f_sh.at[idxs[o].at[pl.ds(j * RB, RB)]],
                        accs[o], gsems[o],
                    ).wait()
                pltpu.async_copy(
                    accs[o], h1p_hbm.at[pl.ds(bp * RB, RB)], wsems[o]
                )

    estart = wid * BPW2

    @pl.loop(0, NCHUNK2)
    def _(ci):
        pltpu.sync_copy(dst_hbm.at[pl.ds(estart + ci * CHUNK2, CHUNK2)], dstc)

        @pl.loop(0, CHUNK2 // 2)
        def _(pi):
            for half in range(2):
                b = pi * 2 + half

                @pl.when(b >= 2)
                def _():
                    pltpu.make_async_copy(
                        ones_v, deg_sh.at[dstc.at[0]], ssems[half]
                    ).wait()

                pltpu.async_copy(
                    ones_v, deg_sh.at[dstc.at[b]], ssems[half], add=True
                )

        pltpu.make_async_copy(ones_v, deg_sh.at[dstc.at[0]], ssems[0]).wait()
        pltpu.make_async_copy(ones_v, deg_sh.at[dstc.at[0]], ssems[1]).wait()

    pltpu.make_async_copy(acc0, h1p_hbm.at[pl.ds(0, RB)], wsem0).wait()
    pltpu.make_async_copy(acc1, h1p_hbm.at[pl.ds(0, RB)], wsem1).wait()

    plsc.subcore_barrier()
    _dump_core_slice(deg_sh, deg_hbm.at[cid], sid)


def _k3_body(h1p_hbm, src_hbm, dst_hbm, agg_hbm,
             srcc, dstc, rows0, rows1, acc_sh, gsem0, gsem1, ssem0, ssem1):
    cid = lax.axis_index("c")
    sid = lax.axis_index("s")
    wid = sid * 2 + cid

    _zero2d(rows0, EB, HID)
    z0 = sid * 640
    for k in range(640 // EB):
        pltpu.sync_copy(rows0, acc_sh.at[pl.ds(z0 + k * EB, EB)])
    plsc.subcore_barrier()

    _agg_loop(h1p_hbm, src_hbm, dst_hbm, acc_sh, srcc, dstc, (rows0, rows1),
              (gsem0, gsem1), (ssem0, ssem1), wid * BPW, CHUNK, NCHUNK)

    plsc.subcore_barrier()
    _dump_core_slice(acc_sh, agg_hbm.at[cid], sid)


def _k5_body(h2_hbm, src_hbm, dst_hbm, agg_hbm,
             srcc, dstc, rows0, rows1, z16, h2_sh, acc_sh,
             gsem0, gsem1, ssem0, ssem1):
    cid = lax.axis_index("c")
    sid = lax.axis_index("s")
    wid = sid * 2 + cid

    _zero2d(z16, 128, 16)
    z0 = sid * 640
    for k in range(5):
        pltpu.sync_copy(z16, acc_sh.at[pl.ds(z0 + k * 128, 128)])
    pltpu.sync_copy(h2_hbm.at[pl.ds(sid * 625, 625)], h2_sh.at[pl.ds(sid * 625, 625)])
    plsc.subcore_barrier()

    _agg_loop(h2_sh, src_hbm, dst_hbm, acc_sh, srcc, dstc, (rows0, rows1),
              (gsem0, gsem1), (ssem0, ssem1), wid * BPW2, CHUNK2, NCHUNK2)

    plsc.subcore_barrier()
    _dump_core_slice(acc_sh, agg_hbm.at[cid], sid)


def _k4_body(h1p_ref, agg_ref, deg_ref, w2_ref, b2_ref, h2_ref):
    norm = deg_ref[0, :, 0:1] + deg_ref[1, :, 0:1] + 1.0
    h1 = jnp.maximum((h1p_ref[...] + agg_ref[0] + agg_ref[1]) / norm, 0.0)
    h2_ref[...] = (
        jnp.dot(h1, w2_ref[...], preferred_element_type=_f32) + b2_ref[...]
    )


def _k6_body(h2_ref, agg_ref, deg_ref, m_ref,
             w1_ref, b1_ref, w2_ref, b2_ref, w3_ref, b3_ref, out_ref):
    norm = deg_ref[0, :, 0:1] + deg_ref[1, :, 0:1] + 1.0
    ge = (h2_ref[...] + agg_ref[0] + agg_ref[1]) / norm
    x = jnp.maximum(jnp.dot(ge, w1_ref[...], preferred_element_type=_f32) + b1_ref[...], 0.0)
    x = jnp.maximum(jnp.dot(x, w2_ref[...], preferred_element_type=_f32) + b2_ref[...], 0.0)
    logits = jnp.dot(x, w3_ref[...], preferred_element_type=_f32) + b3_ref[...]
    sel = jnp.where(m_ref[...] > 0.5, logits, -jnp.inf)
    mx = jnp.max(sel)
    e = jnp.exp(sel - mx)
    out_ref[...] = e / jnp.sum(e)


def kernel(real_features, cat_features, edge_index, mask,
           emb0, emb1, emb2, emb3, W1, b1, W2, b2,
           fc1_w, fc1_b, fc2_w, fc2_b, fc3_w, fc3_b):
    cat = cat_features.astype(_i32)
    src = edge_index[0].astype(_i32)
    dst = edge_index[1].astype(_i32)

    src2d = src.reshape(NBLK, EB)
    dst2d = dst.reshape(NBLK, EB)
    npad = NBLK2 * EB2 - E
    pad_src = (jnp.arange(npad, dtype=_i32) * 131) % N
    pad_dst = N + (jnp.arange(npad, dtype=_i32) % 240)
    src2d_w = jnp.concatenate([src, pad_src]).reshape(NBLK2, EB2)
    dst2d_w = jnp.concatenate([dst, pad_dst]).reshape(NBLK2, EB2)

    cat_off = cat + (jnp.arange(NCAT, dtype=_i32) * VOCAB)[None, :]
    cat_flat = cat_off.reshape(NRB, RB, NCAT).transpose(0, 2, 1).reshape(-1)

    W1r = W1[:HID]
    W1e = W1[HID:].reshape(NCAT, 64, HID)
    embs = jnp.stack([emb0, emb1, emb2, emb3])
    b1r = b1.reshape(1, HID)
    b2r = b2.reshape(1, OUT)
    maskf = mask.astype(_f32).reshape(N, 1)

    F = pl.pallas_call(
        _k1_fold,
        grid=(NCAT,),
        in_specs=[
            pl.BlockSpec((1, VOCAB, 64), lambda j: (j, 0, 0)),
            pl.BlockSpec((1, 64, HID), lambda j: (j, 0, 0)),
        ],
        out_specs=pl.BlockSpec((1, VOCAB, HID), lambda j: (j, 0, 0)),
        out_shape=jax.ShapeDtypeStruct((NCAT, VOCAB, HID), _f32),
    )(embs, W1e)

    RBLK = 1000
    R = pl.pallas_call(
        _k1_dense,
        grid=(N // RBLK,),
        in_specs=[
            pl.BlockSpec((RBLK, HID), lambda i: (i, 0)),
            pl.BlockSpec((HID, HID), lambda i: (0, 0)),
            pl.BlockSpec((1, HID), lambda i: (0, 0)),
        ],
        out_specs=pl.BlockSpec((RBLK, HID), lambda i: (i, 0)),
        out_shape=jax.ShapeDtypeStruct((N, HID), _f32),
    )(real_features, W1r, b1r)

    h1p, deg = pl.kernel(
        _k2_body,
        out_type=(
            jax.ShapeDtypeStruct((N, HID), _f32),
            jax.ShapeDtypeStruct((2, N, 16), _f32),
        ),
        mesh=_mesh,
        compiler_params=pltpu.CompilerParams(use_tc_tiling_on_sc=False),
        scratch_types=[
            pltpu.VMEM((NCAT * RB,), _i32),
            pltpu.VMEM((NCAT * RB,), _i32),
            pltpu.VMEM((RB, HID), _f32),
            pltpu.VMEM((RB, HID), _f32),
            pltpu.VMEM((CHUNK2, EB2), _i32),
            pltpu.VMEM((EB2, 16), _f32),
            pltpu.VMEM((128, 16), _f32),
            pltpu.VMEM_SHARED((NCAT * VOCAB, HID), _f32),
            pltpu.VMEM_SHARED((PADN, 16), _f32),
            pltpu.SemaphoreType.DMA,
            pltpu.SemaphoreType.DMA,
            pltpu.SemaphoreType.DMA,
            pltpu.SemaphoreType.DMA,
            pltpu.SemaphoreType.DMA,
            pltpu.SemaphoreType.DMA,
        ],
    )(R, F.reshape(NCAT * VOCAB, HID), cat_flat, dst2d_w)

    agg1 = pl.kernel(
        _k3_body,
        out_type=jax.ShapeDtypeStruct((2, N, HID), _f32),
        mesh=_mesh,
        compiler_params=pltpu.CompilerParams(use_tc_tiling_on_sc=False),
        scratch_types=[
            pltpu.VMEM((CHUNK, EB), _i32),
            pltpu.VMEM((CHUNK, EB), _i32),
            pltpu.VMEM((EB, HID), _f32),
            pltpu.VMEM((EB, HID), _f32),
            pltpu.VMEM_SHARED((PADN, HID), _f32),
            pltpu.SemaphoreType.DMA,
            pltpu.SemaphoreType.DMA,
            pltpu.SemaphoreType.DMA,
            pltpu.SemaphoreType.DMA,
        ],
    )(h1p, src2d, dst2d)

    h2 = pl.pallas_call(
        _k4_body,
        grid=(N // RBLK,),
        in_specs=[
            pl.BlockSpec((RBLK, HID), lambda i: (i, 0)),
            pl.BlockSpec((2, RBLK, HID), lambda i: (0, i, 0)),
            pl.BlockSpec((2, RBLK, 16), lambda i: (0, i, 0)),
            pl.BlockSpec((HID, OUT), lambda i: (0, 0)),
            pl.BlockSpec((1, OUT), lambda i: (0, 0)),
        ],
        out_specs=pl.BlockSpec((RBLK, OUT), lambda i: (i, 0)),
        out_shape=jax.ShapeDtypeStruct((N, OUT), _f32),
    )(h1p, agg1, deg, W2, b2r)

    agg2 = pl.kernel(
        _k5_body,
        out_type=jax.ShapeDtypeStruct((2, N, OUT), _f32),
        mesh=_mesh,
        compiler_params=pltpu.CompilerParams(use_tc_tiling_on_sc=False),
        scratch_types=[
            pltpu.VMEM((CHUNK2, EB2), _i32),
            pltpu.VMEM((CHUNK2, EB2), _i32),
            pltpu.VMEM((EB2, OUT), _f32),
            pltpu.VMEM((EB2, OUT), _f32),
            pltpu.VMEM((128, 16), _f32),
            pltpu.VMEM_SHARED((N, OUT), _f32),
            pltpu.VMEM_SHARED((PADN, OUT), _f32),
            pltpu.SemaphoreType.DMA,
            pltpu.SemaphoreType.DMA,
            pltpu.SemaphoreType.DMA,
            pltpu.SemaphoreType.DMA,
        ],
    )(h2, src2d_w, dst2d_w)

    probs = pl.pallas_call(
        _k6_body,
        out_shape=jax.ShapeDtypeStruct((N, 1), _f32),
    )(h2, agg2, deg, maskf,
      fc1_w, fc1_b.reshape(1, 24), fc2_w, fc2_b.reshape(1, 24),
      fc3_w, fc3_b.reshape(1, 1))

    return probs.reshape(-1)
